# baseline (speedup 1.0000x reference)
import functools

import jax
import jax.numpy as jnp
from jax import lax
from jax.experimental import pallas as pl
from jax.experimental.pallas import tpu as pltpu
from jax.experimental.pallas import tpu_sc as plsc

N_NODES = 10000
N_EDGES = 160000
FEATS = 256
HALF = 128

IDX_CHUNK = 128
SBLK = 128
N_SBLKS = N_EDGES // SBLK
GBLK = 256
GSUB = GBLK // IDX_CHUNK
N_GBLKS = N_EDGES // GBLK
NC = 2
NS = 16
ROW_CHUNK = 200
N_ROW_CHUNKS = N_NODES // ROW_CHUNK

_sc_mesh = plsc.VectorSubcoreMesh(core_axis_name="c", subcore_axis_name="s")


def _segment_sum_sc(m, dst3, zeros_half):
    @functools.partial(
        pl.kernel,
        out_type=jax.ShapeDtypeStruct((N_NODES, FEATS), jnp.float32),
        mesh=_sc_mesh,
        scratch_types=[
            pltpu.VMEM_SHARED((N_NODES, HALF), jnp.float32),
        ],
    )
    def k(m_hbm, dst_hbm, zeros_hbm, h_hbm, acc_sh):
        c = lax.axis_index("c")
        s = lax.axis_index("s")
        n_row_iter = (N_ROW_CHUNKS + NS - 1) // NS

        @pl.loop(0, n_row_iter)
        def _(i):
            rchunk = i * NS + s

            @pl.when(rchunk < N_ROW_CHUNKS)
            def _():
                r0 = rchunk * ROW_CHUNK
                pltpu.sync_copy(
                    zeros_hbm.at[pl.ds(r0, ROW_CHUNK)],
                    acc_sh.at[pl.ds(r0, ROW_CHUNK)],
                )

        plsc.subcore_barrier()

        def body(idx_v, rows_v):
            pltpu.sync_copy(rows_v, acc_sh.at[idx_v.at[0, 0]], add=True)

        def run_pipeline(col_blk):
            pltpu.emit_pipeline(
                body,
                grid=(N_SBLKS,),
                in_specs=[
                    pl.BlockSpec((1, 1, IDX_CHUNK), lambda i: (i, 0, 0)),
                    pl.BlockSpec(
                        (SBLK, HALF), lambda i: (i, col_blk)
                    ),
                ],
                out_specs=[],
                core_axis_name="s",
                dimension_semantics=(pltpu.PARALLEL,),
            )(dst_hbm, m_hbm)

        run_pipeline(c)

        plsc.subcore_barrier()

        @pl.loop(0, n_row_iter)
        def _(i):
            rchunk = i * NS + s

            @pl.when(rchunk < N_ROW_CHUNKS)
            def _():
                r0 = rchunk * ROW_CHUNK
                pltpu.sync_copy(
                    acc_sh.at[pl.ds(r0, ROW_CHUNK)],
                    h_hbm.at[pl.ds(r0, ROW_CHUNK), pl.ds(c * HALF, HALF)],
                )

    return k(m, dst3, zeros_half)


def _gather_rows_sc(table_i32, idx3):
    @functools.partial(
        pl.kernel,
        out_type=jax.ShapeDtypeStruct((N_EDGES, HALF), jnp.int32),
        mesh=_sc_mesh,
    )
    def k(tab_hbm, idx_hbm, g_hbm):
        def body(idx_v, rows_v):
            for j in range(GSUB):
                pltpu.sync_copy(
                    tab_hbm.at[idx_v.at[0, j]],
                    rows_v.at[pl.ds(j * IDX_CHUNK, IDX_CHUNK)],
                )

        pltpu.emit_pipeline(
            body,
            grid=(N_GBLKS,),
            in_specs=[
                pl.BlockSpec((1, GSUB, IDX_CHUNK), lambda i: (i, 0, 0))
            ],
            out_specs=[
                pl.BlockSpec((GBLK, HALF), lambda i: (i, 0))
            ],
            core_axis_name=("c", "s"),
            dimension_semantics=(pltpu.PARALLEL,),
        )(idx_hbm, g_hbm)

    return k(table_i32, idx3)


_NODE_BLK = 1000


def _bf16_bits(x):
    i = jax.lax.bitcast_convert_type(x, jnp.int32)
    rounded = i + 0x7FFF + (jax.lax.shift_right_logical(i, 16) & 1)
    return rounded & jnp.int32(-65536)


def _node_tc(h, W_top, W_node, b_edge, b_node):
    def body(h_ref, wt_ref, wn_ref, be_ref, bn_ref, tab_ref, hout_ref):
        hb = h_ref[...]
        hwt = (
            jnp.dot(hb, wt_ref[...], preferred_element_type=jnp.float32)
            + be_ref[...]
        )
        lo = jax.lax.shift_right_logical(_bf16_bits(hwt[:, :HALF]), 16)
        hi = _bf16_bits(hwt[:, HALF:])
        tab_ref[...] = lo | hi
        hout_ref[...] = jnp.maximum(
            jnp.dot(hb, wn_ref[...], preferred_element_type=jnp.float32)
            + bn_ref[...],
            0.0,
        )

    full = pl.BlockSpec((FEATS, FEATS), lambda i: (0, 0))
    bias = pl.BlockSpec((1, FEATS), lambda i: (0, 0))
    blk = pl.BlockSpec((_NODE_BLK, FEATS), lambda i: (i, 0))
    halfblk = pl.BlockSpec((_NODE_BLK, HALF), lambda i: (i, 0))
    return pl.pallas_call(
        body,
        grid=(N_NODES // _NODE_BLK,),
        in_specs=[blk, full, full, bias, bias],
        out_specs=[halfblk, blk],
        out_shape=[
            jax.ShapeDtypeStruct((N_NODES, HALF), jnp.int32),
            jax.ShapeDtypeStruct((N_NODES, FEATS), jnp.float32),
        ],
    )(h, W_top, W_node, b_edge, b_node)


_EDGE_BLK = 1000


def _matmul_pack_tc(m, W_bot):
    def body(m_ref, w_ref, o_ref):
        mm = jnp.dot(
            m_ref[...], w_ref[...], preferred_element_type=jnp.float32
        )
        lo = jax.lax.shift_right_logical(_bf16_bits(mm[:, :HALF]), 16)
        hi = _bf16_bits(mm[:, HALF:])
        o_ref[...] = lo | hi

    full = pl.BlockSpec((FEATS, FEATS), lambda i: (0, 0))
    blk = pl.BlockSpec((_EDGE_BLK, FEATS), lambda i: (i, 0))
    halfblk = pl.BlockSpec((_EDGE_BLK, HALF), lambda i: (i, 0))
    return pl.pallas_call(
        body,
        grid=(N_EDGES // _EDGE_BLK,),
        in_specs=[blk, full],
        out_specs=halfblk,
        out_shape=jax.ShapeDtypeStruct((N_EDGES, HALF), jnp.int32),
    )(m, W_bot)


def _unpack(w):
    lo = jax.lax.bitcast_convert_type(jax.lax.shift_left(w, 16), jnp.float32)
    hi = jax.lax.bitcast_convert_type(w & jnp.int32(-65536), jnp.float32)
    return lo, hi


def _final_tc(mWb_i32, g_i32):
    def body(a_ref, g_ref, o_ref):
        a_lo, a_hi = _unpack(a_ref[...])
        g_lo, g_hi = _unpack(g_ref[...])
        o_ref[:, :HALF] = jnp.maximum(a_lo + g_lo, 0.0)
        o_ref[:, HALF:] = jnp.maximum(a_hi + g_hi, 0.0)

    blk = pl.BlockSpec((_EDGE_BLK, FEATS), lambda i: (i, 0))
    halfblk = pl.BlockSpec((_EDGE_BLK, HALF), lambda i: (i, 0))
    return pl.pallas_call(
        body,
        grid=(N_EDGES // _EDGE_BLK,),
        in_specs=[halfblk, halfblk],
        out_specs=blk,
        out_shape=jax.ShapeDtypeStruct((N_EDGES, FEATS), jnp.float32),
    )(mWb_i32, g_i32)


def kernel(m, edge_index, W_node, b_node, W_edge, b_edge):
    src3 = edge_index[0].astype(jnp.int32).reshape(N_GBLKS, GSUB, IDX_CHUNK)
    dst3 = edge_index[1].astype(jnp.int32).reshape(N_SBLKS, 1, IDX_CHUNK)
    zeros_half = jnp.zeros((N_NODES, HALF), jnp.float32)

    mWb_i32 = _matmul_pack_tc(m, W_edge[FEATS:])
    h = _segment_sum_sc(m, dst3, zeros_half)
    tab_i32, h_out = _node_tc(
        h,
        W_edge[:FEATS],
        W_node,
        b_edge.reshape(1, FEATS),
        b_node.reshape(1, FEATS),
    )
    g_i32 = _gather_rows_sc(tab_i32, src3)
    m_out = _final_tc(mWb_i32, g_i32)
    return (m_out, h_out)


# speedup vs baseline: 1.4101x; 1.4101x over previous
import functools

import jax
import jax.numpy as jnp
from jax import lax
from jax.experimental import pallas as pl
from jax.experimental.pallas import tpu as pltpu
from jax.experimental.pallas import tpu_sc as plsc

N_NODES = 10000
N_EDGES = 160000
FEATS = 256
HALF = 128

IDX_CHUNK = 128
SBLK = 128
N_SBLKS = N_EDGES // SBLK
GBLK = 256
GSUB = GBLK // IDX_CHUNK
N_GBLKS = N_EDGES // GBLK
NC = 2
NS = 16
ROW_CHUNK = 200
N_ROW_CHUNKS = N_NODES // ROW_CHUNK

_sc_mesh = plsc.VectorSubcoreMesh(core_axis_name="c", subcore_axis_name="s")


def _segment_sum_sc(m, dst3, zeros_half):
    @functools.partial(
        pl.kernel,
        out_type=jax.ShapeDtypeStruct((N_NODES, FEATS), jnp.float32),
        mesh=_sc_mesh,
        scratch_types=[
            pltpu.VMEM_SHARED((N_NODES, HALF), jnp.float32),
        ],
    )
    def k(m_hbm, dst_hbm, zeros_hbm, h_hbm, acc_sh):
        c = lax.axis_index("c")
        s = lax.axis_index("s")
        n_row_iter = (N_ROW_CHUNKS + NS - 1) // NS

        @pl.loop(0, n_row_iter)
        def _(i):
            rchunk = i * NS + s

            @pl.when(rchunk < N_ROW_CHUNKS)
            def _():
                r0 = rchunk * ROW_CHUNK
                pltpu.sync_copy(
                    zeros_hbm.at[pl.ds(r0, ROW_CHUNK)],
                    acc_sh.at[pl.ds(r0, ROW_CHUNK)],
                )

        plsc.subcore_barrier()

        def body(idx_v, rows_v):
            pltpu.sync_copy(rows_v, acc_sh.at[idx_v.at[0, 0]], add=True)

        def run_pipeline(col_blk):
            pltpu.emit_pipeline(
                body,
                grid=(N_SBLKS,),
                in_specs=[
                    pl.BlockSpec((1, 1, IDX_CHUNK), lambda i: (i, 0, 0)),
                    pl.BlockSpec(
                        (SBLK, HALF), lambda i: (i, col_blk)
                    ),
                ],
                out_specs=[],
                core_axis_name="s",
                dimension_semantics=(pltpu.PARALLEL,),
            )(dst_hbm, m_hbm)

        run_pipeline(c)

        plsc.subcore_barrier()

        @pl.loop(0, n_row_iter)
        def _(i):
            rchunk = i * NS + s

            @pl.when(rchunk < N_ROW_CHUNKS)
            def _():
                r0 = rchunk * ROW_CHUNK
                pltpu.sync_copy(
                    acc_sh.at[pl.ds(r0, ROW_CHUNK)],
                    h_hbm.at[pl.ds(r0, ROW_CHUNK), pl.ds(c * HALF, HALF)],
                )

    return k(m, dst3, zeros_half)


def _gather_rows_sc(table_i32, idx3):
    @functools.partial(
        pl.kernel,
        out_type=jax.ShapeDtypeStruct((N_EDGES, HALF), jnp.int32),
        mesh=_sc_mesh,
    )
    def k(tab_hbm, idx_hbm, g_hbm):
        def body(idx_v, rows_v):
            for j in range(GSUB):
                pltpu.sync_copy(
                    tab_hbm.at[idx_v.at[0, j]],
                    rows_v.at[pl.ds(j * IDX_CHUNK, IDX_CHUNK)],
                )

        pltpu.emit_pipeline(
            body,
            grid=(N_GBLKS,),
            in_specs=[
                pl.BlockSpec((1, GSUB, IDX_CHUNK), lambda i: (i, 0, 0))
            ],
            out_specs=[
                pl.BlockSpec((GBLK, HALF), lambda i: (i, 0))
            ],
            core_axis_name=("c", "s"),
            dimension_semantics=(pltpu.PARALLEL,),
        )(idx_hbm, g_hbm)

    return k(table_i32, idx3)


_NODE_BLK = 2000


def _bf16_bits(x):
    i = jax.lax.bitcast_convert_type(x, jnp.int32)
    rounded = i + 0x7FFF + (jax.lax.shift_right_logical(i, 16) & 1)
    return rounded & jnp.int32(-65536)


def _node_tc(h, W_top, W_node, b_edge, b_node):
    def body(h_ref, wt_ref, wn_ref, be_ref, bn_ref, tab_ref, hout_ref):
        hb = h_ref[...]
        hwt = (
            jnp.dot(hb, wt_ref[...], preferred_element_type=jnp.float32)
            + be_ref[...]
        )
        lo = jax.lax.shift_right_logical(_bf16_bits(hwt[:, :HALF]), 16)
        hi = _bf16_bits(hwt[:, HALF:])
        tab_ref[...] = lo | hi
        hout_ref[...] = jnp.maximum(
            jnp.dot(hb, wn_ref[...], preferred_element_type=jnp.float32)
            + bn_ref[...],
            0.0,
        )

    full = pl.BlockSpec((FEATS, FEATS), lambda i: (0, 0))
    bias = pl.BlockSpec((1, FEATS), lambda i: (0, 0))
    blk = pl.BlockSpec((_NODE_BLK, FEATS), lambda i: (i, 0))
    halfblk = pl.BlockSpec((_NODE_BLK, HALF), lambda i: (i, 0))
    return pl.pallas_call(
        body,
        grid=(N_NODES // _NODE_BLK,),
        in_specs=[blk, full, full, bias, bias],
        out_specs=[halfblk, blk],
        out_shape=[
            jax.ShapeDtypeStruct((N_NODES, HALF), jnp.int32),
            jax.ShapeDtypeStruct((N_NODES, FEATS), jnp.float32),
        ],
    )(h, W_top, W_node, b_edge, b_node)


_EDGE_BLK = 2000


def _unpack(w):
    lo = jax.lax.bitcast_convert_type(jax.lax.shift_left(w, 16), jnp.float32)
    hi = jax.lax.bitcast_convert_type(w & jnp.int32(-65536), jnp.float32)
    return lo, hi


def _edge_tc(m, W_bot, g_i32):
    def body(m_ref, w_ref, g_ref, o_ref):
        mm = jnp.dot(
            m_ref[...], w_ref[...], preferred_element_type=jnp.float32
        )
        g_lo, g_hi = _unpack(g_ref[...])
        o_ref[:, :HALF] = jnp.maximum(mm[:, :HALF] + g_lo, 0.0)
        o_ref[:, HALF:] = jnp.maximum(mm[:, HALF:] + g_hi, 0.0)

    full = pl.BlockSpec((FEATS, FEATS), lambda i: (0, 0))
    blk = pl.BlockSpec((_EDGE_BLK, FEATS), lambda i: (i, 0))
    halfblk = pl.BlockSpec((_EDGE_BLK, HALF), lambda i: (i, 0))
    return pl.pallas_call(
        body,
        grid=(N_EDGES // _EDGE_BLK,),
        in_specs=[blk, full, halfblk],
        out_specs=blk,
        out_shape=jax.ShapeDtypeStruct((N_EDGES, FEATS), jnp.float32),
    )(m, W_bot, g_i32)


def kernel(m, edge_index, W_node, b_node, W_edge, b_edge):
    src3 = edge_index[0].astype(jnp.int32).reshape(N_GBLKS, GSUB, IDX_CHUNK)
    dst3 = edge_index[1].astype(jnp.int32).reshape(N_SBLKS, 1, IDX_CHUNK)
    zeros_half = jnp.zeros((N_NODES, HALF), jnp.float32)

    h = _segment_sum_sc(m, dst3, zeros_half)
    tab_i32, h_out = _node_tc(
        h,
        W_edge[:FEATS],
        W_node,
        b_edge.reshape(1, FEATS),
        b_node.reshape(1, FEATS),
    )
    g_i32 = _gather_rows_sc(tab_i32, src3)
    m_out = _edge_tc(m, W_edge[FEATS:], g_i32)
    return (m_out, h_out)


# speedup vs baseline: 1.4539x; 1.0310x over previous
import functools

import jax
import jax.numpy as jnp
from jax import lax
from jax.experimental import pallas as pl
from jax.experimental.pallas import tpu as pltpu
from jax.experimental.pallas import tpu_sc as plsc

N_NODES = 10000
N_EDGES = 160000
FEATS = 256
HALF = 128

IDX_CHUNK = 128
SBLK = 128
N_SBLKS = N_EDGES // SBLK
GBLK = 256
GSUB = GBLK // IDX_CHUNK
N_GBLKS = N_EDGES // GBLK
NC = 2
NS = 16
ROW_CHUNK = 200
N_ROW_CHUNKS = N_NODES // ROW_CHUNK

_sc_mesh = plsc.VectorSubcoreMesh(core_axis_name="c", subcore_axis_name="s")


def _segment_sum_sc(m, dst3, zeros_half):
    @functools.partial(
        pl.kernel,
        out_type=jax.ShapeDtypeStruct((N_NODES, FEATS), jnp.float32),
        mesh=_sc_mesh,
        scratch_types=[
            pltpu.VMEM_SHARED((N_NODES, HALF), jnp.float32),
        ],
    )
    def k(m_hbm, dst_hbm, zeros_hbm, h_hbm, acc_sh):
        c = lax.axis_index("c")
        s = lax.axis_index("s")
        n_row_iter = (N_ROW_CHUNKS + NS - 1) // NS

        @pl.loop(0, n_row_iter)
        def _(i):
            rchunk = i * NS + s

            @pl.when(rchunk < N_ROW_CHUNKS)
            def _():
                r0 = rchunk * ROW_CHUNK
                pltpu.sync_copy(
                    zeros_hbm.at[pl.ds(r0, ROW_CHUNK)],
                    acc_sh.at[pl.ds(r0, ROW_CHUNK)],
                )

        plsc.subcore_barrier()

        def body(idx_v, rows_v):
            pltpu.sync_copy(rows_v, acc_sh.at[idx_v.at[0, 0]], add=True)

        def run_pipeline(col_blk):
            pltpu.emit_pipeline(
                body,
                grid=(N_SBLKS,),
                in_specs=[
                    pl.BlockSpec((1, 1, IDX_CHUNK), lambda i: (i, 0, 0)),
                    pl.BlockSpec(
                        (SBLK, HALF), lambda i: (i, col_blk)
                    ),
                ],
                out_specs=[],
                core_axis_name="s",
                dimension_semantics=(pltpu.PARALLEL,),
            )(dst_hbm, m_hbm)

        run_pipeline(c)

        plsc.subcore_barrier()

        @pl.loop(0, n_row_iter)
        def _(i):
            rchunk = i * NS + s

            @pl.when(rchunk < N_ROW_CHUNKS)
            def _():
                r0 = rchunk * ROW_CHUNK
                pltpu.sync_copy(
                    acc_sh.at[pl.ds(r0, ROW_CHUNK)],
                    h_hbm.at[pl.ds(r0, ROW_CHUNK), pl.ds(c * HALF, HALF)],
                )

    return k(m, dst3, zeros_half)


def _gather_rows_sc(table_i32, idx3):
    @functools.partial(
        pl.kernel,
        out_type=jax.ShapeDtypeStruct((N_EDGES, HALF), jnp.int32),
        mesh=_sc_mesh,
    )
    def k(tab_hbm, idx_hbm, g_hbm):
        def body(idx_v, rows_v):
            for j in range(GSUB):
                pltpu.sync_copy(
                    tab_hbm.at[idx_v.at[0, j]],
                    rows_v.at[pl.ds(j * IDX_CHUNK, IDX_CHUNK)],
                )

        pltpu.emit_pipeline(
            body,
            grid=(N_GBLKS,),
            in_specs=[
                pl.BlockSpec((1, GSUB, IDX_CHUNK), lambda i: (i, 0, 0))
            ],
            out_specs=[
                pl.BlockSpec((GBLK, HALF), lambda i: (i, 0))
            ],
            core_axis_name=("c", "s"),
            dimension_semantics=(pltpu.PARALLEL,),
        )(idx_hbm, g_hbm)

    return k(table_i32, idx3)


_NODE_BLK = 2000


def _bf16_bits(x):
    i = jax.lax.bitcast_convert_type(x, jnp.int32)
    rounded = i + 0x7FFF + (jax.lax.shift_right_logical(i, 16) & 1)
    return rounded & jnp.int32(-65536)


def _node_tc(h, W_top, W_node, b_edge, b_node):
    def body(h_ref, wt_ref, wn_ref, be_ref, bn_ref, tab_ref, hout_ref):
        hb = h_ref[...]
        hwt = (
            jnp.dot(hb, wt_ref[...], preferred_element_type=jnp.float32)
            + be_ref[...]
        )
        lo = jax.lax.shift_right_logical(_bf16_bits(hwt[:, :HALF]), 16)
        hi = _bf16_bits(hwt[:, HALF:])
        tab_ref[...] = lo | hi
        hout_ref[...] = jnp.maximum(
            jnp.dot(hb, wn_ref[...], preferred_element_type=jnp.float32)
            + bn_ref[...],
            0.0,
        )

    full = pl.BlockSpec((FEATS, FEATS), lambda i: (0, 0))
    bias = pl.BlockSpec((1, FEATS), lambda i: (0, 0))
    blk = pl.BlockSpec((_NODE_BLK, FEATS), lambda i: (i, 0))
    halfblk = pl.BlockSpec((_NODE_BLK, HALF), lambda i: (i, 0))
    return pl.pallas_call(
        body,
        grid=(N_NODES // _NODE_BLK,),
        in_specs=[blk, full, full, bias, bias],
        out_specs=[halfblk, blk],
        out_shape=[
            jax.ShapeDtypeStruct((N_NODES, HALF), jnp.int32),
            jax.ShapeDtypeStruct((N_NODES, FEATS), jnp.float32),
        ],
    )(h, W_top, W_node, b_edge, b_node)


_EDGE_BLK = 4000


def _unpack(w):
    lo = jax.lax.bitcast_convert_type(jax.lax.shift_left(w, 16), jnp.float32)
    hi = jax.lax.bitcast_convert_type(w & jnp.int32(-65536), jnp.float32)
    return lo, hi


def _edge_tc(m, W_bot, g_i32):
    def body(m_ref, w_ref, g_ref, o_ref):
        mm = jnp.dot(
            m_ref[...], w_ref[...], preferred_element_type=jnp.float32
        )
        g_lo, g_hi = _unpack(g_ref[...])
        o_ref[:, :HALF] = jnp.maximum(mm[:, :HALF] + g_lo, 0.0)
        o_ref[:, HALF:] = jnp.maximum(mm[:, HALF:] + g_hi, 0.0)

    full = pl.BlockSpec((FEATS, FEATS), lambda i: (0, 0))
    blk = pl.BlockSpec((_EDGE_BLK, FEATS), lambda i: (i, 0))
    halfblk = pl.BlockSpec((_EDGE_BLK, HALF), lambda i: (i, 0))
    return pl.pallas_call(
        body,
        grid=(N_EDGES // _EDGE_BLK,),
        in_specs=[blk, full, halfblk],
        out_specs=blk,
        out_shape=jax.ShapeDtypeStruct((N_EDGES, FEATS), jnp.float32),
    )(m, W_bot, g_i32)


def kernel(m, edge_index, W_node, b_node, W_edge, b_edge):
    src3 = edge_index[0].astype(jnp.int32).reshape(N_GBLKS, GSUB, IDX_CHUNK)
    dst3 = edge_index[1].astype(jnp.int32).reshape(N_SBLKS, 1, IDX_CHUNK)
    zeros_half = jnp.zeros((N_NODES, HALF), jnp.float32)

    h = _segment_sum_sc(m, dst3, zeros_half)
    tab_i32, h_out = _node_tc(
        h,
        W_edge[:FEATS],
        W_node,
        b_edge.reshape(1, FEATS),
        b_node.reshape(1, FEATS),
    )
    g_i32 = _gather_rows_sc(tab_i32, src3)
    m_out = _edge_tc(m, W_edge[FEATS:], g_i32)
    return (m_out, h_out)


# speedup vs baseline: 1.4570x; 1.0021x over previous
import functools

import jax
import jax.numpy as jnp
from jax import lax
from jax.experimental import pallas as pl
from jax.experimental.pallas import tpu as pltpu
from jax.experimental.pallas import tpu_sc as plsc

N_NODES = 10000
N_EDGES = 160000
FEATS = 256
HALF = 128

IDX_CHUNK = 128
SBLK = 128
N_SBLKS = N_EDGES // SBLK
GBLK = 256
GSUB = GBLK // IDX_CHUNK
N_GBLKS = N_EDGES // GBLK
NC = 2
NS = 16
ROW_CHUNK = 200
N_ROW_CHUNKS = N_NODES // ROW_CHUNK

_sc_mesh = plsc.VectorSubcoreMesh(core_axis_name="c", subcore_axis_name="s")


def _segment_sum_sc(m, dst3, zeros_half):
    @functools.partial(
        pl.kernel,
        out_type=jax.ShapeDtypeStruct((N_NODES, FEATS), jnp.float32),
        mesh=_sc_mesh,
        scratch_types=[
            pltpu.VMEM_SHARED((N_NODES, HALF), jnp.float32),
        ],
    )
    def k(m_hbm, dst_hbm, zeros_hbm, h_hbm, acc_sh):
        c = lax.axis_index("c")
        s = lax.axis_index("s")
        n_row_iter = (N_ROW_CHUNKS + NS - 1) // NS

        @pl.loop(0, n_row_iter)
        def _(i):
            rchunk = i * NS + s

            @pl.when(rchunk < N_ROW_CHUNKS)
            def _():
                r0 = rchunk * ROW_CHUNK
                pltpu.sync_copy(
                    zeros_hbm.at[pl.ds(r0, ROW_CHUNK)],
                    acc_sh.at[pl.ds(r0, ROW_CHUNK)],
                )

        plsc.subcore_barrier()

        def body(idx_v, rows_v):
            pltpu.sync_copy(rows_v, acc_sh.at[idx_v.at[0, 0]], add=True)

        def run_pipeline(col_blk):
            pltpu.emit_pipeline(
                body,
                grid=(N_SBLKS,),
                in_specs=[
                    pl.BlockSpec((1, 1, IDX_CHUNK), lambda i: (i, 0, 0)),
                    pl.BlockSpec(
                        (SBLK, HALF), lambda i: (i, col_blk)
                    ),
                ],
                out_specs=[],
                core_axis_name="s",
                dimension_semantics=(pltpu.PARALLEL,),
            )(dst_hbm, m_hbm)

        run_pipeline(c)

        plsc.subcore_barrier()

        @pl.loop(0, n_row_iter)
        def _(i):
            rchunk = i * NS + s

            @pl.when(rchunk < N_ROW_CHUNKS)
            def _():
                r0 = rchunk * ROW_CHUNK
                pltpu.sync_copy(
                    acc_sh.at[pl.ds(r0, ROW_CHUNK)],
                    h_hbm.at[pl.ds(r0, ROW_CHUNK), pl.ds(c * HALF, HALF)],
                )

    return k(m, dst3, zeros_half)


def _gather_rows_sc(table_i32, idx3):
    @functools.partial(
        pl.kernel,
        out_type=jax.ShapeDtypeStruct((N_EDGES, HALF), jnp.int32),
        mesh=_sc_mesh,
    )
    def k(tab_hbm, idx_hbm, g_hbm):
        def body(idx_v, rows_v):
            for j in range(GSUB):
                pltpu.sync_copy(
                    tab_hbm.at[idx_v.at[0, j]],
                    rows_v.at[pl.ds(j * IDX_CHUNK, IDX_CHUNK)],
                )

        pltpu.emit_pipeline(
            body,
            grid=(N_GBLKS,),
            in_specs=[
                pl.BlockSpec((1, GSUB, IDX_CHUNK), lambda i: (i, 0, 0))
            ],
            out_specs=[
                pl.BlockSpec((GBLK, HALF), lambda i: (i, 0))
            ],
            core_axis_name=("c", "s"),
            dimension_semantics=(pltpu.PARALLEL,),
        )(idx_hbm, g_hbm)

    return k(table_i32, idx3)


_NODE_BLK = 2000


def _bf16_bits(x):
    i = jax.lax.bitcast_convert_type(x, jnp.int32)
    rounded = i + 0x7FFF + (jax.lax.shift_right_logical(i, 16) & 1)
    return rounded & jnp.int32(-65536)


def _node_tc(h, W_top, W_node, b_edge, b_node):
    def body(h_ref, wt_ref, wn_ref, be_ref, bn_ref, tab_ref, hout_ref):
        hb = h_ref[...]
        hwt = (
            jnp.dot(hb, wt_ref[...], preferred_element_type=jnp.float32)
            + be_ref[...]
        )
        lo = jax.lax.shift_right_logical(_bf16_bits(hwt[:, :HALF]), 16)
        hi = _bf16_bits(hwt[:, HALF:])
        tab_ref[...] = lo | hi
        hout_ref[...] = jnp.maximum(
            jnp.dot(hb, wn_ref[...], preferred_element_type=jnp.float32)
            + bn_ref[...],
            0.0,
        )

    full = pl.BlockSpec((FEATS, FEATS), lambda i: (0, 0))
    bias = pl.BlockSpec((1, FEATS), lambda i: (0, 0))
    blk = pl.BlockSpec((_NODE_BLK, FEATS), lambda i: (i, 0))
    halfblk = pl.BlockSpec((_NODE_BLK, HALF), lambda i: (i, 0))
    return pl.pallas_call(
        body,
        grid=(N_NODES // _NODE_BLK,),
        in_specs=[blk, full, full, bias, bias],
        out_specs=[halfblk, blk],
        out_shape=[
            jax.ShapeDtypeStruct((N_NODES, HALF), jnp.int32),
            jax.ShapeDtypeStruct((N_NODES, FEATS), jnp.float32),
        ],
    )(h, W_top, W_node, b_edge, b_node)


_EDGE_BLK = 8000


def _unpack(w):
    lo = jax.lax.bitcast_convert_type(jax.lax.shift_left(w, 16), jnp.float32)
    hi = jax.lax.bitcast_convert_type(w & jnp.int32(-65536), jnp.float32)
    return lo, hi


def _edge_tc(m, W_bot, g_i32):
    def body(m_ref, w_ref, g_ref, o_ref):
        mm = jnp.dot(
            m_ref[...], w_ref[...], preferred_element_type=jnp.float32
        )
        g_lo, g_hi = _unpack(g_ref[...])
        o_ref[:, :HALF] = jnp.maximum(mm[:, :HALF] + g_lo, 0.0)
        o_ref[:, HALF:] = jnp.maximum(mm[:, HALF:] + g_hi, 0.0)

    full = pl.BlockSpec((FEATS, FEATS), lambda i: (0, 0))
    blk = pl.BlockSpec((_EDGE_BLK, FEATS), lambda i: (i, 0))
    halfblk = pl.BlockSpec((_EDGE_BLK, HALF), lambda i: (i, 0))
    return pl.pallas_call(
        body,
        grid=(N_EDGES // _EDGE_BLK,),
        in_specs=[blk, full, halfblk],
        out_specs=blk,
        out_shape=jax.ShapeDtypeStruct((N_EDGES, FEATS), jnp.float32),
    )(m, W_bot, g_i32)


def kernel(m, edge_index, W_node, b_node, W_edge, b_edge):
    src3 = edge_index[0].astype(jnp.int32).reshape(N_GBLKS, GSUB, IDX_CHUNK)
    dst3 = edge_index[1].astype(jnp.int32).reshape(N_SBLKS, 1, IDX_CHUNK)
    zeros_half = jnp.zeros((N_NODES, HALF), jnp.float32)

    h = _segment_sum_sc(m, dst3, zeros_half)
    tab_i32, h_out = _node_tc(
        h,
        W_edge[:FEATS],
        W_node,
        b_edge.reshape(1, FEATS),
        b_node.reshape(1, FEATS),
    )
    g_i32 = _gather_rows_sc(tab_i32, src3)
    m_out = _edge_tc(m, W_edge[FEATS:], g_i32)
    return (m_out, h_out)


# speedup vs baseline: 1.4670x; 1.0069x over previous
import functools

import jax
import jax.numpy as jnp
from jax import lax
from jax.experimental import pallas as pl
from jax.experimental.pallas import tpu as pltpu
from jax.experimental.pallas import tpu_sc as plsc

N_NODES = 10000
N_EDGES = 160000
FEATS = 256
HALF = 128

IDX_CHUNK = 128
SBLK = 128
N_SBLKS = N_EDGES // SBLK
GBLK = 128
N_GBLKS = N_EDGES // GBLK
HALF_EDGES = N_EDGES // 2
N_HBLKS = N_GBLKS // 2
NC = 2
NS = 16
ROW_CHUNK = 200
N_ROW_CHUNKS = N_NODES // ROW_CHUNK

_sc_mesh = plsc.VectorSubcoreMesh(core_axis_name="c", subcore_axis_name="s")


def _segment_sum_sc(m, dst3, zeros_half):
    @functools.partial(
        pl.kernel,
        out_type=jax.ShapeDtypeStruct((N_NODES, FEATS), jnp.float32),
        mesh=_sc_mesh,
        scratch_types=[
            pltpu.VMEM_SHARED((N_NODES, HALF), jnp.float32),
        ],
    )
    def k(m_hbm, dst_hbm, zeros_hbm, h_hbm, acc_sh):
        c = lax.axis_index("c")
        s = lax.axis_index("s")
        n_row_iter = (N_ROW_CHUNKS + NS - 1) // NS

        @pl.loop(0, n_row_iter)
        def _(i):
            rchunk = i * NS + s

            @pl.when(rchunk < N_ROW_CHUNKS)
            def _():
                r0 = rchunk * ROW_CHUNK
                pltpu.sync_copy(
                    zeros_hbm.at[pl.ds(r0, ROW_CHUNK)],
                    acc_sh.at[pl.ds(r0, ROW_CHUNK)],
                )

        plsc.subcore_barrier()

        def body(idx_v, rows_v):
            pltpu.sync_copy(rows_v, acc_sh.at[idx_v.at[0, 0]], add=True)

        def run_pipeline(col_blk):
            pltpu.emit_pipeline(
                body,
                grid=(N_SBLKS,),
                in_specs=[
                    pl.BlockSpec((1, 1, IDX_CHUNK), lambda i: (i, 0, 0)),
                    pl.BlockSpec(
                        (SBLK, HALF), lambda i: (i, col_blk)
                    ),
                ],
                out_specs=[],
                core_axis_name="s",
                dimension_semantics=(pltpu.PARALLEL,),
            )(dst_hbm, m_hbm)

        run_pipeline(c)

        plsc.subcore_barrier()

        @pl.loop(0, n_row_iter)
        def _(i):
            rchunk = i * NS + s

            @pl.when(rchunk < N_ROW_CHUNKS)
            def _():
                r0 = rchunk * ROW_CHUNK
                pltpu.sync_copy(
                    acc_sh.at[pl.ds(r0, ROW_CHUNK)],
                    h_hbm.at[pl.ds(r0, ROW_CHUNK), pl.ds(c * HALF, HALF)],
                )

    return k(m, dst3, zeros_half)


def _gather_rows_sc(table_i32, idx3):
    @functools.partial(
        pl.kernel,
        out_type=jax.ShapeDtypeStruct((HALF_EDGES, HALF), jnp.int32),
        mesh=_sc_mesh,
    )
    def k(tab_hbm, idx_hbm, g_hbm):
        def body(idx_v, rows_v):
            pltpu.sync_copy(tab_hbm.at[idx_v.at[0, 0]], rows_v)

        pltpu.emit_pipeline(
            body,
            grid=(N_HBLKS,),
            in_specs=[
                pl.BlockSpec((1, 1, IDX_CHUNK), lambda i: (i, 0, 0))
            ],
            out_specs=[
                pl.BlockSpec((GBLK, HALF), lambda i: (i, 0))
            ],
            core_axis_name=("c", "s"),
            dimension_semantics=(pltpu.PARALLEL,),
        )(idx_hbm, g_hbm)

    return k(table_i32, idx3)


_NODE_BLK = 2000


def _bf16_bits(x):
    i = jax.lax.bitcast_convert_type(x, jnp.int32)
    rounded = i + 0x7FFF + (jax.lax.shift_right_logical(i, 16) & 1)
    return rounded & jnp.int32(-65536)


def _node_tc(h, W_top, W_node, b_edge, b_node):
    def body(h_ref, wt_ref, wn_ref, be_ref, bn_ref, tab_ref, hout_ref):
        hb = h_ref[...]
        hwt = (
            jnp.dot(hb, wt_ref[...], preferred_element_type=jnp.float32)
            + be_ref[...]
        )
        lo = jax.lax.shift_right_logical(_bf16_bits(hwt[:, :HALF]), 16)
        hi = _bf16_bits(hwt[:, HALF:])
        tab_ref[...] = lo | hi
        hout_ref[...] = jnp.maximum(
            jnp.dot(hb, wn_ref[...], preferred_element_type=jnp.float32)
            + bn_ref[...],
            0.0,
        )

    full = pl.BlockSpec((FEATS, FEATS), lambda i: (0, 0))
    bias = pl.BlockSpec((1, FEATS), lambda i: (0, 0))
    blk = pl.BlockSpec((_NODE_BLK, FEATS), lambda i: (i, 0))
    halfblk = pl.BlockSpec((_NODE_BLK, HALF), lambda i: (i, 0))
    return pl.pallas_call(
        body,
        grid=(N_NODES // _NODE_BLK,),
        in_specs=[blk, full, full, bias, bias],
        out_specs=[halfblk, blk],
        out_shape=[
            jax.ShapeDtypeStruct((N_NODES, HALF), jnp.int32),
            jax.ShapeDtypeStruct((N_NODES, FEATS), jnp.float32),
        ],
    )(h, W_top, W_node, b_edge, b_node)


_EDGE_BLK = 8000


def _unpack(w):
    lo = jax.lax.bitcast_convert_type(jax.lax.shift_left(w, 16), jnp.float32)
    hi = jax.lax.bitcast_convert_type(w & jnp.int32(-65536), jnp.float32)
    return lo, hi


def _edge_half_tc(m, W_bot, g_half, off_blocks, buf=None):
    def body(m_ref, w_ref, g_ref, *rest):
        o_ref = rest[-1]
        mm = jnp.dot(
            m_ref[...], w_ref[...], preferred_element_type=jnp.float32
        )
        g_lo, g_hi = _unpack(g_ref[...])
        o_ref[:, :HALF] = jnp.maximum(mm[:, :HALF] + g_lo, 0.0)
        o_ref[:, HALF:] = jnp.maximum(mm[:, HALF:] + g_hi, 0.0)

    full = pl.BlockSpec((FEATS, FEATS), lambda i: (0, 0))
    blk = pl.BlockSpec((_EDGE_BLK, FEATS), lambda i: (i + off_blocks, 0))
    halfblk = pl.BlockSpec((_EDGE_BLK, HALF), lambda i: (i, 0))
    in_specs = [blk, full, halfblk]
    args = [m, W_bot, g_half]
    aliases = {}
    if buf is not None:
        in_specs.append(pl.BlockSpec(memory_space=pl.ANY))
        args.append(buf)
        aliases = {3: 0}
    return pl.pallas_call(
        body,
        grid=(HALF_EDGES // _EDGE_BLK,),
        in_specs=in_specs,
        out_specs=blk,
        out_shape=jax.ShapeDtypeStruct((N_EDGES, FEATS), jnp.float32),
        input_output_aliases=aliases,
    )(*args)


def kernel(m, edge_index, W_node, b_node, W_edge, b_edge):
    src3 = edge_index[0].astype(jnp.int32).reshape(N_GBLKS, 1, IDX_CHUNK)
    dst3 = edge_index[1].astype(jnp.int32).reshape(N_SBLKS, 1, IDX_CHUNK)
    zeros_half = jnp.zeros((N_NODES, HALF), jnp.float32)

    h = _segment_sum_sc(m, dst3, zeros_half)
    tab_i32, h_out = _node_tc(
        h,
        W_edge[:FEATS],
        W_node,
        b_edge.reshape(1, FEATS),
        b_node.reshape(1, FEATS),
    )
    W_bot = W_edge[FEATS:]
    g0 = _gather_rows_sc(tab_i32, src3[:N_HBLKS])
    buf = _edge_half_tc(m, W_bot, g0, 0)
    g1 = _gather_rows_sc(tab_i32, src3[N_HBLKS:])
    m_out = _edge_half_tc(m, W_bot, g1, HALF_EDGES // _EDGE_BLK, buf)
    return (m_out, h_out)
